# baseline (speedup 1.0000x reference)
import functools

import jax
import jax.numpy as jnp
from jax import lax
from jax.experimental import pallas as pl
from jax.experimental.pallas import tpu as pltpu
from jax.experimental.pallas import tpu_sc as plsc

_R = 2
_NBUF = 4
_NIDX = 8
_IDX_AHEAD = 6
_G_AHEAD = 3


def kernel(tokens, wte_weight, learned_embedding):
    B, S = tokens.shape
    V, D = wte_weight.shape
    P = learned_embedding.shape[0]
    tok = tokens.astype(jnp.int32)

    info = plsc.get_sparse_core_info()
    NC, NS = info.num_cores, info.num_subcores
    NW = NC * NS
    assert B % (NW * _R) == 0
    n_chunks = B // (NW * _R)
    assert n_chunks % _NIDX == 0 and n_chunks // _NIDX >= 3

    mesh = plsc.VectorSubcoreMesh(core_axis_name="c", subcore_axis_name="s")

    @functools.partial(
        pl.kernel,
        mesh=mesh,
        out_type=jax.ShapeDtypeStruct((B, S, D), jnp.float32),
        scratch_types=(
            [pltpu.VMEM((_R, S), jnp.int32) for _ in range(_NIDX)]
            + [pltpu.VMEM((_R, S, D), jnp.float32) for _ in range(_NBUF)]
            + [
                pltpu.VMEM((P, D), jnp.float32),
                pltpu.SemaphoreType.DMA((_NIDX,)),
                pltpu.SemaphoreType.DMA((_NBUF,)),
                pltpu.SemaphoreType.DMA((_NBUF,)),
            ]
        ),
        compiler_params=pltpu.CompilerParams(use_tc_tiling_on_sc=False),
    )
    def run(tok_hbm, wte_hbm, learned_hbm, out_hbm, *scratch):
        idx_v = scratch[:_NIDX]
        buf_v = scratch[_NIDX : _NIDX + _NBUF]
        le_v = scratch[_NIDX + _NBUF]
        sem_i, sem_g, sem_o = scratch[_NIDX + _NBUF + 1 :]
        wid = lax.axis_index("s") * NC + lax.axis_index("c")
        chunk0 = wid * n_chunks

        pltpu.sync_copy(learned_hbm, le_v)
        le_regs = [
            [le_v[r, pl.ds(c * 16, 16)] for c in range(D // 16)] for r in range(P)
        ]

        def idx_desc(c, s):
            return pltpu.make_async_copy(
                tok_hbm.at[pl.ds((chunk0 + c) * _R, _R)], idx_v[s], sem_i.at[s]
            )

        def gather_descs(s, p):
            descs = []
            for r in range(_R):
                for off, ln in ((0, 128), (128, S - 128)):
                    descs.append(
                        pltpu.make_async_copy(
                            wte_hbm.at[idx_v[s].at[r, pl.ds(off, ln)]],
                            buf_v[p].at[r, pl.ds(off, ln)],
                            sem_g.at[p],
                        )
                    )
            return descs

        def write_desc(c, p):
            return pltpu.make_async_copy(
                buf_v[p], out_hbm.at[pl.ds((chunk0 + c) * _R, _R)], sem_o.at[p]
            )

        def fix_prefix(p):
            for r in range(_R):
                for rr in range(P):
                    for c in range(D // 16):
                        buf_v[p][r, rr, pl.ds(c * 16, 16)] = le_regs[rr][c]

        for c in range(_IDX_AHEAD):
            idx_desc(c, c % _NIDX).start()
        for c in range(_G_AHEAD):
            idx_desc(c, c % _NIDX).wait()
            for d in gather_descs(c % _NIDX, c % _NBUF):
                d.start()

        def phase(c, k, wait_prev_write, start_idx, start_gather):
            p = k % _NBUF
            for g in gather_descs(k, p):
                g.wait()
            fix_prefix(p)
            write_desc(c, p).start()
            if start_idx:
                idx_desc(c + _IDX_AHEAD, (k + _IDX_AHEAD) % _NIDX).start()
            if wait_prev_write:
                write_desc(c - 1, (k - 1) % _NBUF).wait()
            if start_gather:
                sg = (k + _G_AHEAD) % _NIDX
                idx_desc(c + _G_AHEAD, sg).wait()
                for g in gather_descs(sg, (k + _G_AHEAD) % _NBUF):
                    g.start()

        for c in range(_NIDX):
            phase(c, c, c >= 1, c + _IDX_AHEAD < n_chunks, c + _G_AHEAD < n_chunks)

        def body(j, carry):
            c0 = j * _NIDX
            for k in range(_NIDX):
                phase(c0 + k, k, True, True, True)
            return carry

        lax.fori_loop(1, n_chunks // _NIDX - 1, body, 0)

        c0 = n_chunks - _NIDX
        for k in range(_NIDX):
            c = c0 + k
            phase(c, k, True, c + _IDX_AHEAD < n_chunks, c + _G_AHEAD < n_chunks)
        write_desc(n_chunks - 1, (_NIDX - 1) % _NBUF).wait()

    return run(tok, wte_weight, learned_embedding)
